# baseline (speedup 1.0000x reference)
import jax
import jax.numpy as jnp
from jax import lax
from jax.experimental import pallas as pl
from jax.experimental.pallas import tpu as pltpu

_K = 8192
_D = 256
_TOK_BLK = 1024


def _vq_body(zn_ref, z_ref, w_ref, wn_ref, out_ref):
    z = z_ref[0].astype(jnp.bfloat16)
    w = w_ref[...]
    dot2 = lax.dot_general(
        z, w, (((0,), (1,)), ((), ())),
        preferred_element_type=jnp.float32,
    )
    dist = (zn_ref[...] - dot2) + wn_ref[...]
    idx = jnp.argmin(dist, axis=1).astype(jnp.int32)
    out_ref[0, 0, :] = idx


def kernel(z_e_x, W):
    B, C, H, Wd = z_e_x.shape
    n_tok = B * H * Wd
    z3 = z_e_x.reshape(B, C, H * Wd)
    znorm = jnp.sum(z_e_x * z_e_x, axis=1).reshape(n_tok, 1)
    wnorm = jnp.sum(W * W, axis=1)[None, :]
    W_bf = (2.0 * W).astype(jnp.bfloat16)

    n_blk = n_tok // _TOK_BLK
    idx = pl.pallas_call(
        _vq_body,
        grid=(n_blk,),
        in_specs=[
            pl.BlockSpec((_TOK_BLK, 1), lambda i: (i, 0)),
            pl.BlockSpec((1, _D, _TOK_BLK), lambda i: (i, 0, 0)),
            pl.BlockSpec((_K, _D), lambda i: (0, 0)),
            pl.BlockSpec((1, _K), lambda i: (0, 0)),
        ],
        out_specs=pl.BlockSpec((1, 1, _TOK_BLK), lambda i: (i, 0, 0)),
        out_shape=jax.ShapeDtypeStruct((n_blk, 1, _TOK_BLK), jnp.int32),
        compiler_params=pltpu.CompilerParams(
            dimension_semantics=("parallel",)),
    )(znorm, z3, W_bf, wnorm)
    return idx.reshape(B, H, Wd)


# speedup vs baseline: 1.0937x; 1.0937x over previous
import jax
import jax.numpy as jnp
from jax import lax
from jax.experimental import pallas as pl
from jax.experimental.pallas import tpu as pltpu

_K = 8192
_D = 256
_TOK_BLK = 1024


def _vq_body(zn_ref, z_ref, w_ref, wn_ref, out_ref):
    z = z_ref[...]
    w = w_ref[...]
    dot2 = lax.dot_general(
        z, w, (((1,), (1,)), ((), ())),
        preferred_element_type=jnp.float32,
    )
    dist = (zn_ref[...] - dot2) + wn_ref[...]
    idx = jnp.argmin(dist, axis=1).astype(jnp.int32)
    out_ref[0, 0, :] = idx


def kernel(z_e_x, W):
    B, C, H, Wd = z_e_x.shape
    n_tok = B * H * Wd
    znorm = jnp.sum(z_e_x * z_e_x, axis=1).reshape(n_tok, 1)
    wnorm = jnp.sum(W * W, axis=1)[None, :]
    flat_bf = jnp.transpose(z_e_x.astype(jnp.bfloat16),
                            (0, 2, 3, 1)).reshape(n_tok, C)
    W_bf = (2.0 * W).astype(jnp.bfloat16)

    n_blk = n_tok // _TOK_BLK
    idx = pl.pallas_call(
        _vq_body,
        grid=(n_blk,),
        in_specs=[
            pl.BlockSpec((_TOK_BLK, 1), lambda i: (i, 0)),
            pl.BlockSpec((_TOK_BLK, _D), lambda i: (i, 0)),
            pl.BlockSpec((_K, _D), lambda i: (0, 0)),
            pl.BlockSpec((1, _K), lambda i: (0, 0)),
        ],
        out_specs=pl.BlockSpec((1, 1, _TOK_BLK), lambda i: (i, 0, 0)),
        out_shape=jax.ShapeDtypeStruct((n_blk, 1, _TOK_BLK), jnp.int32),
        compiler_params=pltpu.CompilerParams(
            dimension_semantics=("parallel",)),
    )(znorm, flat_bf, W_bf, wnorm)
    return idx.reshape(B, H, Wd)
